# baseline (speedup 1.0000x reference)
import functools

import jax
import jax.numpy as jnp
from jax import lax
from jax.experimental import pallas as pl
from jax.experimental.pallas import tpu as pltpu
from jax.experimental.pallas import tpu_sc as plsc

N = 10000
E = 160000
G = 64

NC = 2
NS = 16
EPAD = 163840
_K = 128
_F = 128
_NPA = 624
_NPL = N - (NS - 1) * _NPA


_BE = 6400
_BN = 2000


def _edge_mlp_body(eaT_ref, w1_ref, b1_ref, w2_ref, b2_ref, out_ref):
    eaT = eaT_ref[...]
    h = jnp.dot(w1_ref[...].T, eaT, preferred_element_type=jnp.float32)
    h = jnp.maximum(h + b1_ref[...].T, 0.0)
    z = jnp.dot(w2_ref[...].T, h, preferred_element_type=jnp.float32)
    out_ref[...] = jax.nn.sigmoid(z + b2_ref[0, 0])


def _edge_mlp(eaT, w1, b1, w2, b2):
    return pl.pallas_call(
        _edge_mlp_body,
        grid=(E // _BE,),
        in_specs=[
            pl.BlockSpec((3, _BE), lambda i: (0, i)),
            pl.BlockSpec((3, 64), lambda i: (0, 0)),
            pl.BlockSpec((1, 64), lambda i: (0, 0)),
            pl.BlockSpec((64, 1), lambda i: (0, 0)),
            pl.BlockSpec((1, 1), lambda i: (0, 0), memory_space=pltpu.SMEM),
        ],
        out_specs=pl.BlockSpec((1, _BE), lambda i: (0, i)),
        out_shape=jax.ShapeDtypeStruct((1, E), jnp.float32),
    )(eaT, w1, b1[None, :], w2, b2[None, :])


def _emb_body(x_ref, w1_ref, b1_ref, w2_ref, b2_ref, out_ref):
    h = jnp.dot(x_ref[...], w1_ref[...], preferred_element_type=jnp.float32)
    h = jnp.maximum(h + b1_ref[...], 0.0)
    h = jnp.dot(h, w2_ref[...], preferred_element_type=jnp.float32)
    out_ref[...] = jnp.maximum(h + b2_ref[...], 0.0)


def _emb(x, w1, b1, w2, b2):
    return pl.pallas_call(
        _emb_body,
        grid=(N // _BN,),
        in_specs=[
            pl.BlockSpec((_BN, 40), lambda i: (i, 0)),
            pl.BlockSpec((40, 512), lambda i: (0, 0)),
            pl.BlockSpec((1, 512), lambda i: (0, 0)),
            pl.BlockSpec((512, 512), lambda i: (0, 0)),
            pl.BlockSpec((1, 512), lambda i: (0, 0)),
        ],
        out_specs=pl.BlockSpec((_BN, 512), lambda i: (i, 0)),
        out_shape=jax.ShapeDtypeStruct((N, 512), jnp.float32),
    )(x, w1, b1[None, :], w2, b2[None, :])


def _conv_post_body(agg_ref, h_ref, d2_ref, w_ref, b_ref, out_ref):
    a = agg_ref[...] + d2_ref[...] * h_ref[...]
    z = jnp.dot(a, w_ref[...], preferred_element_type=jnp.float32)
    out_ref[...] = jnp.maximum(z + b_ref[...], 0.0)


def _conv_post(agg, h, dinv2, w, b):
    cin, cout = w.shape
    bn = 1000
    return pl.pallas_call(
        _conv_post_body,
        grid=(N // bn,),
        in_specs=[
            pl.BlockSpec((bn, cin), lambda i: (i, 0)),
            pl.BlockSpec((bn, cin), lambda i: (i, 0)),
            pl.BlockSpec((bn, 1), lambda i: (i, 0)),
            pl.BlockSpec((cin, cout), lambda i: (0, 0)),
            pl.BlockSpec((1, cout), lambda i: (0, 0)),
        ],
        out_specs=pl.BlockSpec((bn, cout), lambda i: (i, 0)),
        out_shape=jax.ShapeDtypeStruct((N, cout), jnp.float32),
    )(agg, h, dinv2, w, b[None, :])


def _pool_body(b3d_ref, h_ref, sums_ref, cnt_ref):
    i = pl.program_id(0)
    gids = jax.lax.broadcasted_iota(jnp.int32, (G, _BN), 0)
    onehot = (b3d_ref[0] == gids).astype(jnp.float32)
    s = jnp.dot(onehot, h_ref[...], preferred_element_type=jnp.float32)
    c = jnp.sum(onehot, axis=1, keepdims=True)

    @pl.when(i == 0)
    def _():
        sums_ref[...] = s
        cnt_ref[...] = c

    @pl.when(i > 0)
    def _():
        sums_ref[...] += s
        cnt_ref[...] += c


def _pool(batch3d, h):
    c = h.shape[1]
    return pl.pallas_call(
        _pool_body,
        grid=(N // _BN,),
        in_specs=[
            pl.BlockSpec((1, 1, _BN), lambda i: (i, 0, 0)),
            pl.BlockSpec((_BN, c), lambda i: (i, 0)),
        ],
        out_specs=[
            pl.BlockSpec((G, c), lambda i: (0, 0)),
            pl.BlockSpec((G, 1), lambda i: (0, 0)),
        ],
        out_shape=[
            jax.ShapeDtypeStruct((G, c), jnp.float32),
            jax.ShapeDtypeStruct((G, 1), jnp.float32),
        ],
    )(batch3d, h)


def _head_body(s_ref, c_ref, w1_ref, b1_ref, w2_ref, b2_ref, out_ref):
    g = s_ref[...] / jnp.maximum(c_ref[...], 1.0)
    g = jnp.dot(g, w1_ref[...], preferred_element_type=jnp.float32)
    g = jnp.maximum(g + b1_ref[...], 0.0)
    z = jnp.dot(g, w2_ref[...], preferred_element_type=jnp.float32)
    out_ref[...] = z + b2_ref[...]


def _head(sums, cnt, fc1_w, fc1_b, head_w, head_b):
    return pl.pallas_call(
        _head_body,
        out_shape=jax.ShapeDtypeStruct((G, head_w.shape[1]), jnp.float32),
    )(sums, cnt, fc1_w, fc1_b[None, :], head_w, head_b[None, :])


_MESH = plsc.VectorSubcoreMesh(core_axis_name="c", subcore_axis_name="s",
                               num_cores=NC, num_subcores=NS)
_SC_PARAMS = pltpu.CompilerParams(needs_layout_passes=False)


def _sc_deg_body(dstF, ewF, degp, dstb, ewb, deg_sh, zbuf):
    c = lax.axis_index("c")
    s = lax.axis_index("s")
    w = c * NS + s
    pltpu.sync_copy(dstF.at[w], dstb)
    pltpu.sync_copy(ewF.at[w], ewb)

    @pl.when(s == 0)
    def _():
        def zloop(i, _):
            zbuf[pl.ds(i * 16, 16)] = jnp.zeros((16,), jnp.float32)
            return 0
        lax.fori_loop(0, N // 16, zloop, 0)
        pltpu.sync_copy(zbuf, deg_sh)

    plsc.subcore_barrier()

    def body(j, _):
        pltpu.sync_copy(ewb.at[j], deg_sh.at[dstb.at[j]], add=True)
        return 0
    lax.fori_loop(0, EPAD // (NC * NS) // _K, body, 0)
    plsc.subcore_barrier()

    @pl.when(s == 0)
    def _():
        pltpu.sync_copy(deg_sh, degp.at[c])


def _sc_deg(dstF, ewF):
    nb = EPAD // (NC * NS) // _K
    f = pl.kernel(
        _sc_deg_body,
        out_type=jax.ShapeDtypeStruct((NC, N), jnp.float32),
        mesh=_MESH,
        compiler_params=_SC_PARAMS,
        scratch_types=[
            pltpu.VMEM((nb, _K), jnp.int32),
            pltpu.VMEM((nb, _K), jnp.float32),
            pltpu.VMEM_SHARED((N,), jnp.float32),
            pltpu.VMEM((N,), jnp.float32),
        ],
    )
    return f(dstF.reshape(NC * NS, nb, _K), ewF.reshape(NC * NS, nb, _K))


def _sc_norm_body(srcF, dstF, ewF, dinv, normF, dinvb, srcb, dstb, ewb, normb):
    c = lax.axis_index("c")
    s = lax.axis_index("s")
    w = c * NS + s
    pltpu.sync_copy(dinv, dinvb)
    pltpu.sync_copy(srcF.at[w], srcb)
    pltpu.sync_copy(dstF.at[w], dstb)
    pltpu.sync_copy(ewF.at[w], ewb)

    def body(k, _):
        sl = pl.ds(k * 16, 16)
        ds_ = plsc.load_gather(dinvb, [srcb[sl]])
        dd = plsc.load_gather(dinvb, [dstb[sl]])
        normb[sl] = ds_ * ewb[sl] * dd
        return 0
    lax.fori_loop(0, EPAD // (NC * NS) // 16, body, 0)
    pltpu.sync_copy(normb, normF.at[w])


def _sc_norm(srcF, dstF, ewF, dinv):
    ept = EPAD // (NC * NS)
    f = pl.kernel(
        _sc_norm_body,
        out_type=jax.ShapeDtypeStruct((NC * NS, ept), jnp.float32),
        mesh=_MESH,
        compiler_params=_SC_PARAMS,
        scratch_types=[
            pltpu.VMEM((N,), jnp.float32),
            pltpu.VMEM((ept,), jnp.int32),
            pltpu.VMEM((ept,), jnp.int32),
            pltpu.VMEM((ept,), jnp.float32),
            pltpu.VMEM((ept,), jnp.float32),
        ],
    )
    out = f(srcF.reshape(NC * NS, ept), dstF.reshape(NC * NS, ept),
            ewF.reshape(NC * NS, ept), dinv)
    return out.reshape(EPAD)


def _sc_agg_body(nch, h2, zeros, srcA, dstA, normA, out,
                 acc_sh, rows0, rows1, sji0, sji1, dji0, dji1,
                 nj0, nj1, gs0, gs1, ss0, ss1):
    c = lax.axis_index("c")
    s = lax.axis_index("s")
    nbt = EPAD // NS // _K
    bufs = (rows0, rows1, sji0, sji1, dji0, dji1, nj0, nj1,
            gs0, gs1, ss0, ss1)

    for ci in range(nch // NC):
        for cc in range(NC):
            ch = ci * NC + cc
            if cc == 0:
                @pl.when(c == 0)
                def _():
                    _agg_chunk(ch, h2, zeros, srcA, dstA, normA, out,
                               acc_sh, s, nbt, *bufs)
            else:
                @pl.when(c == 1)
                def _():
                    _agg_chunk(ch, h2, zeros, srcA, dstA, normA, out,
                               acc_sh, s, nbt, *bufs)


def _agg_chunk(ch, h2, zeros, srcA, dstA, normA, out, acc_sh, s, nbt,
               rows0, rows1, sji0, sji1, dji0, dji1, nj0, nj1,
               gs0, gs1, ss0, ss1):
    @pl.when(s < NS - 1)
    def _():
        sl = pl.ds(pl.multiple_of(s * _NPA, 8), _NPA)
        pltpu.sync_copy(zeros.at[sl], acc_sh.at[sl])

    @pl.when(s == NS - 1)
    def _():
        sl = pl.ds((NS - 1) * _NPA, _NPL)
        pltpu.sync_copy(zeros.at[sl], acc_sh.at[sl])

    plsc.subcore_barrier()
    chbase = ch * N

    def start_gather(j, sji, dji, nj, rows, gsem):
        esl = pl.ds(j * _K, _K)
        pltpu.sync_copy(srcA.at[s, esl], sji)
        pltpu.sync_copy(dstA.at[s, esl], dji)
        pltpu.sync_copy(normA.at[s, esl], nj)
        for i in range(_K // 16):
            sl16 = pl.ds(i * 16, 16)
            sji[sl16] = sji[sl16] + chbase
        pltpu.async_copy(h2.at[sji], rows, gsem)

    def scale_scatter(sji, dji, nj, rows, gsem, ssem):
        pltpu.make_async_copy(h2.at[sji], rows, gsem).wait()

        def sbody(k, _):
            nv = nj[pl.ds(k * 16, 16)]
            for r in range(16):
                sc = nv[r]
                for q in range(_F // 16):
                    sl = pl.ds(q * 16, 16)
                    rows[k * 16 + r, sl] = rows[k * 16 + r, sl] * sc
            return 0
        lax.fori_loop(0, _K // 16, sbody, 0)
        pltpu.async_copy(rows, acc_sh.at[dji], ssem, add=True)

    start_gather(0, sji0, dji0, nj0, rows0, gs0)

    def ebody(jj, _):
        b0 = jj * 2
        @pl.when(jj > 0)
        def _():
            pltpu.make_async_copy(rows1, acc_sh.at[dji1], ss1).wait()
        start_gather(b0 + 1, sji1, dji1, nj1, rows1, gs1)
        scale_scatter(sji0, dji0, nj0, rows0, gs0, ss0)

        @pl.when(jj < nbt // 2 - 1)
        def _():
            pltpu.make_async_copy(rows0, acc_sh.at[dji0], ss0).wait()
            start_gather(b0 + 2, sji0, dji0, nj0, rows0, gs0)
        scale_scatter(sji1, dji1, nj1, rows1, gs1, ss1)
        return 0
    lax.fori_loop(0, nbt // 2, ebody, 0)
    pltpu.make_async_copy(rows0, acc_sh.at[dji0], ss0).wait()
    pltpu.make_async_copy(rows1, acc_sh.at[dji1], ss1).wait()
    plsc.subcore_barrier()

    @pl.when(s < NS - 1)
    def _():
        sl = pl.ds(pl.multiple_of(s * _NPA, 8), _NPA)
        pltpu.sync_copy(acc_sh.at[sl], out.at[ch, sl])

    @pl.when(s == NS - 1)
    def _():
        sl = pl.ds((NS - 1) * _NPA, _NPL)
        pltpu.sync_copy(acc_sh.at[sl], out.at[ch, sl])

    plsc.subcore_barrier()


def _sc_agg(h, zeros, srcA, dstA, normA):
    cdim = h.shape[1]
    nch = cdim // _F
    nbt = EPAD // NS // _K
    h2 = h.reshape(N, nch, _F).transpose(1, 0, 2).reshape(nch * N, _F)
    f = pl.kernel(
        functools.partial(_sc_agg_body, nch),
        out_type=jax.ShapeDtypeStruct((nch, N, _F), jnp.float32),
        mesh=_MESH,
        compiler_params=_SC_PARAMS,
        scratch_types=[
            pltpu.VMEM_SHARED((N, _F), jnp.float32),
            pltpu.VMEM((_K, _F), jnp.float32),
            pltpu.VMEM((_K, _F), jnp.float32),
            pltpu.VMEM((_K,), jnp.int32),
            pltpu.VMEM((_K,), jnp.int32),
            pltpu.VMEM((_K,), jnp.int32),
            pltpu.VMEM((_K,), jnp.int32),
            pltpu.VMEM((_K,), jnp.float32),
            pltpu.VMEM((_K,), jnp.float32),
            pltpu.SemaphoreType.DMA,
            pltpu.SemaphoreType.DMA,
            pltpu.SemaphoreType.DMA,
            pltpu.SemaphoreType.DMA,
        ],
    )
    out = f(h2, zeros, srcA.reshape(NS, nbt * _K), dstA.reshape(NS, nbt * _K),
            normA.reshape(NS, nbt * _K))
    return out.transpose(1, 0, 2).reshape(N, cdim)


def _dinv_body(degp_ref, dinv_ref, dinv2_ref):
    deg = degp_ref[0, :] + degp_ref[1, :] + 1.0
    dinv_ref[...] = jax.lax.rsqrt(deg)[None, :]
    dinv2_ref[...] = (1.0 / deg)[None, :]


def _dinv(degp):
    return pl.pallas_call(
        _dinv_body,
        out_shape=[
            jax.ShapeDtypeStruct((1, N), jnp.float32),
            jax.ShapeDtypeStruct((1, N), jnp.float32),
        ],
    )(degp)


def kernel(x, edge_index, edge_attr, batch, emb_w1, emb_b1, emb_w2, emb_b2,
           ep_w1, ep_b1, ep_w2, ep_b2, c6_w, c6_b, c7_w, c7_b, c8_w, c8_b,
           fc1_w, fc1_b, head_w, head_b):
    src = edge_index[0]
    dst = edge_index[1]
    eaT = jnp.squeeze(edge_attr, axis=2).T

    ew = _edge_mlp(eaT, ep_w1, ep_b1, ep_w2, ep_b2)[0]
    h = _emb(x, emb_w1, emb_b1, emb_w2, emb_b2)

    pad = jnp.arange(EPAD - E, dtype=jnp.int32) % N
    src_p = jnp.concatenate([src, pad])
    dst_p = jnp.concatenate([dst, pad])
    ew_p = jnp.concatenate([ew, jnp.zeros((EPAD - E,), jnp.float32)])

    degp = _sc_deg(dst_p, ew_p)
    dinv, dinv2 = _dinv(degp)
    norm_p = _sc_norm(src_p, dst_p, ew_p, dinv[0])
    dinv2 = dinv2.reshape(N, 1)

    nbt = EPAD // NS // _K
    srcA = src_p.reshape(NS, nbt, _K)
    dstA = dst_p.reshape(NS, nbt, _K)
    normA = norm_p.reshape(NS, nbt, _K)
    zeros = jnp.zeros((N, _F), jnp.float32)

    for w, b in ((c6_w, c6_b), (c7_w, c7_b), (c8_w, c8_b)):
        agg = _sc_agg(h, zeros, srcA, dstA, normA)
        h = _conv_post(agg, h, dinv2, w, b)

    sums, cnt = _pool(batch.astype(jnp.int32).reshape(N // _BN, 1, _BN), h)
    return _head(sums, cnt, fc1_w, fc1_b, head_w, head_b)


# speedup vs baseline: 1.3973x; 1.3973x over previous
import functools

import jax
import jax.numpy as jnp
from jax import lax
from jax.experimental import pallas as pl
from jax.experimental.pallas import tpu as pltpu
from jax.experimental.pallas import tpu_sc as plsc

N = 10000
E = 160000
G = 64

NC = 2
NS = 16
EPAD = 163840
_K = 128
_F = 128
_NPA = 624
_NPL = N - (NS - 1) * _NPA


_BE = 6400
_BN = 2000


def _edge_mlp_body(eaT_ref, w1_ref, b1_ref, w2_ref, b2_ref, out_ref):
    eaT = eaT_ref[...]
    h = jnp.dot(w1_ref[...].T, eaT, preferred_element_type=jnp.float32)
    h = jnp.maximum(h + b1_ref[...].T, 0.0)
    z = jnp.dot(w2_ref[...].T, h, preferred_element_type=jnp.float32)
    out_ref[...] = jax.nn.sigmoid(z + b2_ref[0, 0])


def _edge_mlp(eaT, w1, b1, w2, b2):
    return pl.pallas_call(
        _edge_mlp_body,
        grid=(E // _BE,),
        in_specs=[
            pl.BlockSpec((3, _BE), lambda i: (0, i)),
            pl.BlockSpec((3, 64), lambda i: (0, 0)),
            pl.BlockSpec((1, 64), lambda i: (0, 0)),
            pl.BlockSpec((64, 1), lambda i: (0, 0)),
            pl.BlockSpec((1, 1), lambda i: (0, 0), memory_space=pltpu.SMEM),
        ],
        out_specs=pl.BlockSpec((1, _BE), lambda i: (0, i)),
        out_shape=jax.ShapeDtypeStruct((1, E), jnp.float32),
    )(eaT, w1, b1[None, :], w2, b2[None, :])


def _emb_body(x_ref, w1_ref, b1_ref, w2_ref, b2_ref, out_ref):
    h = jnp.dot(x_ref[...], w1_ref[...], preferred_element_type=jnp.float32)
    h = jnp.maximum(h + b1_ref[...], 0.0)
    h = jnp.dot(h, w2_ref[...], preferred_element_type=jnp.float32)
    out_ref[...] = jnp.maximum(h + b2_ref[...], 0.0)


def _emb(x, w1, b1, w2, b2):
    return pl.pallas_call(
        _emb_body,
        grid=(N // _BN,),
        in_specs=[
            pl.BlockSpec((_BN, 40), lambda i: (i, 0)),
            pl.BlockSpec((40, 512), lambda i: (0, 0)),
            pl.BlockSpec((1, 512), lambda i: (0, 0)),
            pl.BlockSpec((512, 512), lambda i: (0, 0)),
            pl.BlockSpec((1, 512), lambda i: (0, 0)),
        ],
        out_specs=pl.BlockSpec((_BN, 512), lambda i: (i, 0)),
        out_shape=jax.ShapeDtypeStruct((N, 512), jnp.float32),
    )(x, w1, b1[None, :], w2, b2[None, :])


def _conv_post_body(agg_ref, h_ref, d2_ref, w_ref, b_ref, out_ref):
    a = agg_ref[...] + d2_ref[...] * h_ref[...]
    z = jnp.dot(a, w_ref[...], preferred_element_type=jnp.float32)
    out_ref[...] = jnp.maximum(z + b_ref[...], 0.0)


def _conv_post(agg, h, dinv2, w, b):
    cin, cout = w.shape
    bn = 1000
    return pl.pallas_call(
        _conv_post_body,
        grid=(N // bn,),
        in_specs=[
            pl.BlockSpec((bn, cin), lambda i: (i, 0)),
            pl.BlockSpec((bn, cin), lambda i: (i, 0)),
            pl.BlockSpec((bn, 1), lambda i: (i, 0)),
            pl.BlockSpec((cin, cout), lambda i: (0, 0)),
            pl.BlockSpec((1, cout), lambda i: (0, 0)),
        ],
        out_specs=pl.BlockSpec((bn, cout), lambda i: (i, 0)),
        out_shape=jax.ShapeDtypeStruct((N, cout), jnp.float32),
    )(agg, h, dinv2, w, b[None, :])


def _pool_body(b3d_ref, h_ref, sums_ref, cnt_ref):
    i = pl.program_id(0)
    gids = jax.lax.broadcasted_iota(jnp.int32, (G, _BN), 0)
    onehot = (b3d_ref[0] == gids).astype(jnp.float32)
    s = jnp.dot(onehot, h_ref[...], preferred_element_type=jnp.float32)
    c = jnp.sum(onehot, axis=1, keepdims=True)

    @pl.when(i == 0)
    def _():
        sums_ref[...] = s
        cnt_ref[...] = c

    @pl.when(i > 0)
    def _():
        sums_ref[...] += s
        cnt_ref[...] += c


def _pool(batch3d, h):
    c = h.shape[1]
    return pl.pallas_call(
        _pool_body,
        grid=(N // _BN,),
        in_specs=[
            pl.BlockSpec((1, 1, _BN), lambda i: (i, 0, 0)),
            pl.BlockSpec((_BN, c), lambda i: (i, 0)),
        ],
        out_specs=[
            pl.BlockSpec((G, c), lambda i: (0, 0)),
            pl.BlockSpec((G, 1), lambda i: (0, 0)),
        ],
        out_shape=[
            jax.ShapeDtypeStruct((G, c), jnp.float32),
            jax.ShapeDtypeStruct((G, 1), jnp.float32),
        ],
    )(batch3d, h)


def _head_body(s_ref, c_ref, w1_ref, b1_ref, w2_ref, b2_ref, out_ref):
    g = s_ref[...] / jnp.maximum(c_ref[...], 1.0)
    g = jnp.dot(g, w1_ref[...], preferred_element_type=jnp.float32)
    g = jnp.maximum(g + b1_ref[...], 0.0)
    z = jnp.dot(g, w2_ref[...], preferred_element_type=jnp.float32)
    out_ref[...] = z + b2_ref[...]


def _head(sums, cnt, fc1_w, fc1_b, head_w, head_b):
    return pl.pallas_call(
        _head_body,
        out_shape=jax.ShapeDtypeStruct((G, head_w.shape[1]), jnp.float32),
    )(sums, cnt, fc1_w, fc1_b[None, :], head_w, head_b[None, :])


_MESH = plsc.VectorSubcoreMesh(core_axis_name="c", subcore_axis_name="s",
                               num_cores=NC, num_subcores=NS)
_SC_PARAMS = pltpu.CompilerParams(needs_layout_passes=False)


def _sc_deg_body(dstF, ewF, degp, dstb, ewb, deg_sh, zbuf):
    c = lax.axis_index("c")
    s = lax.axis_index("s")
    w = c * NS + s
    pltpu.sync_copy(dstF.at[w], dstb)
    pltpu.sync_copy(ewF.at[w], ewb)

    @pl.when(s == 0)
    def _():
        def zloop(i, _):
            zbuf[pl.ds(i * 16, 16)] = jnp.zeros((16,), jnp.float32)
            return 0
        lax.fori_loop(0, N // 16, zloop, 0)
        pltpu.sync_copy(zbuf, deg_sh)

    plsc.subcore_barrier()

    def body(j, _):
        pltpu.sync_copy(ewb.at[j], deg_sh.at[dstb.at[j]], add=True)
        return 0
    lax.fori_loop(0, EPAD // (NC * NS) // _K, body, 0)
    plsc.subcore_barrier()

    @pl.when(s == 0)
    def _():
        pltpu.sync_copy(deg_sh, degp.at[c])


def _sc_deg(dstF, ewF):
    nb = EPAD // (NC * NS) // _K
    f = pl.kernel(
        _sc_deg_body,
        out_type=jax.ShapeDtypeStruct((NC, N), jnp.float32),
        mesh=_MESH,
        compiler_params=_SC_PARAMS,
        scratch_types=[
            pltpu.VMEM((nb, _K), jnp.int32),
            pltpu.VMEM((nb, _K), jnp.float32),
            pltpu.VMEM_SHARED((N,), jnp.float32),
            pltpu.VMEM((N,), jnp.float32),
        ],
    )
    return f(dstF.reshape(NC * NS, nb, _K), ewF.reshape(NC * NS, nb, _K))


def _sc_norm_body(srcF, dstF, ewF, dinv, normF, dinvb, srcb, dstb, ewb, normb):
    c = lax.axis_index("c")
    s = lax.axis_index("s")
    w = c * NS + s
    pltpu.sync_copy(dinv, dinvb)
    pltpu.sync_copy(srcF.at[w], srcb)
    pltpu.sync_copy(dstF.at[w], dstb)
    pltpu.sync_copy(ewF.at[w], ewb)

    def body(k, _):
        sl = pl.ds(k * 16, 16)
        ds_ = plsc.load_gather(dinvb, [srcb[sl]])
        dd = plsc.load_gather(dinvb, [dstb[sl]])
        normb[sl] = ds_ * ewb[sl] * dd
        return 0
    lax.fori_loop(0, EPAD // (NC * NS) // 16, body, 0)
    pltpu.sync_copy(normb, normF.at[w])


def _sc_norm(srcF, dstF, ewF, dinv):
    ept = EPAD // (NC * NS)
    f = pl.kernel(
        _sc_norm_body,
        out_type=jax.ShapeDtypeStruct((NC * NS, ept), jnp.float32),
        mesh=_MESH,
        compiler_params=_SC_PARAMS,
        scratch_types=[
            pltpu.VMEM((N,), jnp.float32),
            pltpu.VMEM((ept,), jnp.int32),
            pltpu.VMEM((ept,), jnp.int32),
            pltpu.VMEM((ept,), jnp.float32),
            pltpu.VMEM((ept,), jnp.float32),
        ],
    )
    out = f(srcF.reshape(NC * NS, ept), dstF.reshape(NC * NS, ept),
            ewF.reshape(NC * NS, ept), dinv)
    return out.reshape(EPAD)


_KB = 64


def _sc_agg_body(nch, h2, zeros, srcA, dstA, normA, out,
                 acc_sh, srcb, dstb, normb, rows0, rows1,
                 sji0, sji1, dji0, dji1, gs0, gs1, ss0, ss1):
    c = lax.axis_index("c")
    s = lax.axis_index("s")
    nbt = EPAD // NS // _KB
    pltpu.sync_copy(srcA.at[s], srcb)
    pltpu.sync_copy(dstA.at[s], dstb)
    pltpu.sync_copy(normA.at[s], normb)
    bufs = (srcb, dstb, normb, rows0, rows1, sji0, sji1, dji0, dji1,
            gs0, gs1, ss0, ss1)

    for ci in range(nch // NC):
        for cc in range(NC):
            ch = ci * NC + cc
            if cc == 0:
                @pl.when(c == 0)
                def _():
                    _agg_chunk(ch, h2, zeros, out, acc_sh, s, nbt, *bufs)
            else:
                @pl.when(c == 1)
                def _():
                    _agg_chunk(ch, h2, zeros, out, acc_sh, s, nbt, *bufs)


def _agg_chunk(ch, h2, zeros, out, acc_sh, s, nbt,
               srcb, dstb, normb, rows0, rows1, sji0, sji1, dji0, dji1,
               gs0, gs1, ss0, ss1):
    @pl.when(s < NS - 1)
    def _():
        sl = pl.ds(pl.multiple_of(s * _NPA, 8), _NPA)
        pltpu.sync_copy(zeros.at[sl], acc_sh.at[sl])

    @pl.when(s == NS - 1)
    def _():
        sl = pl.ds((NS - 1) * _NPA, _NPL)
        pltpu.sync_copy(zeros.at[sl], acc_sh.at[sl])

    plsc.subcore_barrier()
    chbase = ch * N

    def start_gather(j, sji, dji, rows, gsem):
        for i in range(_KB // 16):
            sl16 = pl.ds(i * 16, 16)
            sji[sl16] = srcb[pl.ds(j * _KB + i * 16, 16)] + chbase
            dji[sl16] = dstb[pl.ds(j * _KB + i * 16, 16)]
        pltpu.async_copy(h2.at[sji], rows, gsem)

    def scale_scatter(j, sji, dji, rows, gsem, ssem):
        pltpu.make_async_copy(h2.at[sji], rows, gsem).wait()

        def sbody(k, _):
            nv = normb[pl.ds(j * _KB + k * 16, 16)]
            for r in range(16):
                sc = nv[r]
                for q in range(_F // 16):
                    sl = pl.ds(q * 16, 16)
                    rows[k * 16 + r, sl] = rows[k * 16 + r, sl] * sc
            return 0
        lax.fori_loop(0, _KB // 16, sbody, 0)
        pltpu.async_copy(rows, acc_sh.at[dji], ssem, add=True)

    start_gather(0, sji0, dji0, rows0, gs0)

    def ebody(jj, _):
        b0 = jj * 2
        @pl.when(jj > 0)
        def _():
            pltpu.make_async_copy(rows1, acc_sh.at[dji1], ss1).wait()
        start_gather(b0 + 1, sji1, dji1, rows1, gs1)
        scale_scatter(b0, sji0, dji0, rows0, gs0, ss0)

        @pl.when(jj < nbt // 2 - 1)
        def _():
            pltpu.make_async_copy(rows0, acc_sh.at[dji0], ss0).wait()
            start_gather(b0 + 2, sji0, dji0, rows0, gs0)
        scale_scatter(b0 + 1, sji1, dji1, rows1, gs1, ss1)
        return 0
    lax.fori_loop(0, nbt // 2, ebody, 0)
    pltpu.make_async_copy(rows0, acc_sh.at[dji0], ss0).wait()
    pltpu.make_async_copy(rows1, acc_sh.at[dji1], ss1).wait()
    plsc.subcore_barrier()

    @pl.when(s < NS - 1)
    def _():
        sl = pl.ds(pl.multiple_of(s * _NPA, 8), _NPA)
        pltpu.sync_copy(acc_sh.at[sl], out.at[ch, sl])

    @pl.when(s == NS - 1)
    def _():
        sl = pl.ds((NS - 1) * _NPA, _NPL)
        pltpu.sync_copy(acc_sh.at[sl], out.at[ch, sl])

    plsc.subcore_barrier()


def _sc_agg(h, zeros, srcA, dstA, normA):
    cdim = h.shape[1]
    nch = cdim // _F
    nbt = EPAD // NS // _K
    h2 = h.reshape(N, nch, _F).transpose(1, 0, 2).reshape(nch * N, _F)
    f = pl.kernel(
        functools.partial(_sc_agg_body, nch),
        out_type=jax.ShapeDtypeStruct((nch, N, _F), jnp.float32),
        mesh=_MESH,
        compiler_params=_SC_PARAMS,
        scratch_types=[
            pltpu.VMEM_SHARED((N, _F), jnp.float32),
            pltpu.VMEM((EPAD // NS,), jnp.int32),
            pltpu.VMEM((EPAD // NS,), jnp.int32),
            pltpu.VMEM((EPAD // NS,), jnp.float32),
            pltpu.VMEM((_KB, _F), jnp.float32),
            pltpu.VMEM((_KB, _F), jnp.float32),
            pltpu.VMEM((_KB,), jnp.int32),
            pltpu.VMEM((_KB,), jnp.int32),
            pltpu.VMEM((_KB,), jnp.int32),
            pltpu.VMEM((_KB,), jnp.int32),
            pltpu.SemaphoreType.DMA,
            pltpu.SemaphoreType.DMA,
            pltpu.SemaphoreType.DMA,
            pltpu.SemaphoreType.DMA,
        ],
    )
    out = f(h2, zeros, srcA.reshape(NS, nbt * _K), dstA.reshape(NS, nbt * _K),
            normA.reshape(NS, nbt * _K))
    return out.transpose(1, 0, 2).reshape(N, cdim)


def _dinv_body(degp_ref, dinv_ref, dinv2_ref):
    deg = degp_ref[0, :] + degp_ref[1, :] + 1.0
    dinv_ref[...] = jax.lax.rsqrt(deg)[None, :]
    dinv2_ref[...] = (1.0 / deg)[None, :]


def _dinv(degp):
    return pl.pallas_call(
        _dinv_body,
        out_shape=[
            jax.ShapeDtypeStruct((1, N), jnp.float32),
            jax.ShapeDtypeStruct((1, N), jnp.float32),
        ],
    )(degp)


def kernel(x, edge_index, edge_attr, batch, emb_w1, emb_b1, emb_w2, emb_b2,
           ep_w1, ep_b1, ep_w2, ep_b2, c6_w, c6_b, c7_w, c7_b, c8_w, c8_b,
           fc1_w, fc1_b, head_w, head_b):
    src = edge_index[0]
    dst = edge_index[1]
    eaT = jnp.squeeze(edge_attr, axis=2).T

    ew = _edge_mlp(eaT, ep_w1, ep_b1, ep_w2, ep_b2)[0]
    h = _emb(x, emb_w1, emb_b1, emb_w2, emb_b2)

    pad = jnp.arange(EPAD - E, dtype=jnp.int32) % N
    src_p = jnp.concatenate([src, pad])
    dst_p = jnp.concatenate([dst, pad])
    ew_p = jnp.concatenate([ew, jnp.zeros((EPAD - E,), jnp.float32)])

    degp = _sc_deg(dst_p, ew_p)
    dinv, dinv2 = _dinv(degp)
    norm_p = _sc_norm(src_p, dst_p, ew_p, dinv[0])
    dinv2 = dinv2.reshape(N, 1)

    nbt = EPAD // NS // _K
    srcA = src_p.reshape(NS, nbt, _K)
    dstA = dst_p.reshape(NS, nbt, _K)
    normA = norm_p.reshape(NS, nbt, _K)
    zeros = jnp.zeros((N, _F), jnp.float32)

    for w, b in ((c6_w, c6_b), (c7_w, c7_b), (c8_w, c8_b)):
        agg = _sc_agg(h, zeros, srcA, dstA, normA)
        h = _conv_post(agg, h, dinv2, w, b)

    sums, cnt = _pool(batch.astype(jnp.int32).reshape(N // _BN, 1, _BN), h)
    return _head(sums, cnt, fc1_w, fc1_b, head_w, head_b)
